# baseline (speedup 1.0000x reference)
import jax
import jax.numpy as jnp
from jax import lax
from jax.experimental import pallas as pl
from jax.experimental.pallas import tpu as pltpu

N = 1024
D = 64
IDX = 64
H = 8
L = 3
BR = 128
NT = N // BR
CC = 1024
NC = N // CC


def _fused_kernel(b1_ref, w2_ref, b2_ref, w3_ref, b3_ref,
                  adj_ref, adjd_ref, t_ref, emb_ref, embblk_ref,
                  w1_ref, y_ref, W_ref, b_ref, o_ref,
                  a_s, ad_s, aT_s, adT_s, tg_s, clT_s):
    bf = jnp.bfloat16
    i = pl.program_id(0)
    dn_bb = (((1,), (1,)), ((), ()))

    @pl.when(i == 0)
    def _():
        for m in range(2):
            w1j = w1_ref[m, :, 1 + IDX:]
            clT = lax.dot_general(w1j, emb_ref[...], dn_bb,
                                  preferred_element_type=jnp.float32)
            b1col = jnp.stack([b1_ref[m, k] for k in range(H)]).reshape(H, 1)
            clT_s[pl.ds(m * H, H), :] = (clT + b1col).astype(bf)

    for m in range(2):
        in_ref = adj_ref if m == 0 else adjd_ref
        out_s = a_s if m == 0 else ad_s
        outT_s = aT_s if m == 0 else adT_s
        w1i = w1_ref[m, :, 1:1 + IDX]
        rw = lax.dot_general(embblk_ref[...], w1i, dn_bb,
                             preferred_element_type=jnp.float32)
        rwb = rw.astype(bf)
        clT2 = clT_s[pl.ds(m * H, H), :]
        for c in range(NC):
            cs = slice(c * CC, (c + 1) * CC)
            A = in_ref[:, cs].astype(bf)
            clc = clT2[:, cs]
            h1 = []
            for k in range(H):
                wa_k = w1_ref[m, k, 0].astype(bf)
                h1.append(jnp.maximum(A * wa_k + rwb[:, k:k + 1] + clc[k:k + 1, :],
                                      bf(0.0)))
            h2 = []
            for k2 in range(H):
                acc = h1[0] * w2_ref[m, k2 * H].astype(bf)
                for j in range(1, H):
                    acc = acc + h1[j] * w2_ref[m, k2 * H + j].astype(bf)
                h2.append(jnp.maximum(acc + b2_ref[m, k2].astype(bf), bf(0.0)))
            out = h2[0] * w3_ref[m, 0].astype(bf)
            for j in range(1, H):
                out = out + h2[j] * w3_ref[m, j].astype(bf)
            out = out + b3_ref[m, 0].astype(bf)
            out_s[pl.ds(i * BR, BR), cs] = out
            outT_s[pl.ds(c * CC, CC), pl.ds(i * BR, BR)] = out.T

    tg_s[pl.ds(i, 1)] = jnp.sum(t_ref[...], axis=0, keepdims=True).reshape(1, 1, N)

    @pl.when(i == NT - 1)
    def _():
        x = y_ref[...]
        for l in range(L):
            xb = x.astype(bf)
            p0 = jnp.dot(a_s[...], xb, preferred_element_type=jnp.float32)
            p1 = jnp.dot(aT_s[...], xb, preferred_element_type=jnp.float32)
            p2 = jnp.dot(ad_s[...], xb, preferred_element_type=jnp.float32)
            p3 = jnp.dot(adT_s[...], xb, preferred_element_type=jnp.float32)
            x = (jnp.dot(p0, W_ref[l, 0], preferred_element_type=jnp.float32)
                 + jnp.dot(p1, W_ref[l, 1], preferred_element_type=jnp.float32)
                 + jnp.dot(p2, W_ref[l, 2], preferred_element_type=jnp.float32)
                 + jnp.dot(p3, W_ref[l, 3], preferred_element_type=jnp.float32)
                 + jnp.dot(x, W_ref[l, 4], preferred_element_type=jnp.float32)
                 + b_ref[l].reshape(1, D))
            if l < L - 1:
                x = jnp.maximum(x, 0.0)
        ones8 = jnp.ones((NT, D), jnp.float32)
        tgp = tg_s[...].reshape(NT, N)
        tgm = lax.dot_general(tgp, ones8, (((0,), (0,)), ((), ())),
                              preferred_element_type=jnp.float32)
        o_ref[...] = x * (tgm * (1.0 / N))


def kernel(y, adj, adj_deriv, t_grad, idx_emb, msg_W1, msg_b1, msg_W2, msg_b2,
           msg_W3, msg_b3, gnn_W, gnn_b):
    smem = pl.BlockSpec(memory_space=pltpu.SMEM)
    out = pl.pallas_call(
        _fused_kernel,
        grid=(NT,),
        in_specs=[
            smem, smem, smem, smem, smem,
            pl.BlockSpec((BR, N), lambda i: (i, 0)),
            pl.BlockSpec((BR, N), lambda i: (i, 0)),
            pl.BlockSpec((BR, N), lambda i: (i, 0)),
            pl.BlockSpec((N, IDX), lambda i: (0, 0)),
            pl.BlockSpec((BR, IDX), lambda i: (i, 0)),
            pl.BlockSpec((2, H, 2 * IDX + 1), lambda i: (0, 0, 0)),
            pl.BlockSpec((N, D), lambda i: (0, 0)),
            pl.BlockSpec((L, 5, D, D), lambda i: (0, 0, 0, 0)),
            pl.BlockSpec((L, D), lambda i: (0, 0)),
        ],
        out_specs=pl.BlockSpec((N, D), lambda i: (0, 0)),
        out_shape=jax.ShapeDtypeStruct((N, D), jnp.float32),
        scratch_shapes=[
            pltpu.VMEM((N, N), jnp.bfloat16),
            pltpu.VMEM((N, N), jnp.bfloat16),
            pltpu.VMEM((N, N), jnp.bfloat16),
            pltpu.VMEM((N, N), jnp.bfloat16),
            pltpu.VMEM((NT, 1, N), jnp.float32),
            pltpu.VMEM((2 * H, N), jnp.bfloat16),
        ],
        compiler_params=pltpu.CompilerParams(
            dimension_semantics=("arbitrary",),
        ),
        name="fused_msg_gnn",
    )(msg_b1, msg_W2.reshape(2, H * H), msg_b2, msg_W3.reshape(2, H), msg_b3,
      adj, adj_deriv, t_grad, idx_emb, idx_emb, msg_W1, y, gnn_W, gnn_b)
    return out


# speedup vs baseline: 1.0080x; 1.0080x over previous
import jax
import jax.numpy as jnp
from jax import lax
from jax.experimental import pallas as pl
from jax.experimental.pallas import tpu as pltpu

N = 1024
D = 64
IDX = 64
H = 8
L = 3
BR = 128
NT = N // BR


def _fused_kernel(b1_ref, w2_ref, b2_ref, w3_ref, b3_ref,
                  adj_hbm, adjd_hbm, t_hbm, emb_ref,
                  w1_ref, y_ref, W_ref, b_ref, o_ref,
                  a_s, ad_s, aT_s, adT_s,
                  ab, db, tb, sems):
    bf = jnp.bfloat16
    dn_bb = (((1,), (1,)), ((), ()))

    def start(s):
        slot = s % 2
        rows = pl.ds(s * BR, BR)
        pltpu.make_async_copy(adj_hbm.at[rows, :], ab.at[slot], sems.at[0, slot]).start()
        pltpu.make_async_copy(adjd_hbm.at[rows, :], db.at[slot], sems.at[1, slot]).start()
        pltpu.make_async_copy(t_hbm.at[rows, :], tb.at[slot], sems.at[2, slot]).start()

    def wait(s):
        slot = s % 2
        pltpu.make_async_copy(ab.at[slot], ab.at[slot], sems.at[0, slot]).wait()
        pltpu.make_async_copy(db.at[slot], db.at[slot], sems.at[1, slot]).wait()
        pltpu.make_async_copy(tb.at[slot], tb.at[slot], sems.at[2, slot]).wait()

    clT2 = []
    for m in range(2):
        w1j = w1_ref[m, :, 1 + IDX:]
        clT = lax.dot_general(w1j, emb_ref[...], dn_bb,
                              preferred_element_type=jnp.float32)
        b1col = jnp.stack([b1_ref[m, k] for k in range(H)]).reshape(H, 1)
        clT2.append((clT + b1col).astype(bf))

    start(0)
    tgacc = None
    for s in range(NT):
        if s + 1 < NT:
            start(s + 1)
        wait(s)
        slot = s % 2
        rows = slice(s * BR, (s + 1) * BR)
        embblk = emb_ref[rows, :]
        for m in range(2):
            in_b = ab if m == 0 else db
            out_s = a_s if m == 0 else ad_s
            outT_s = aT_s if m == 0 else adT_s
            w1i = w1_ref[m, :, 1:1 + IDX]
            rw = lax.dot_general(embblk, w1i, dn_bb,
                                 preferred_element_type=jnp.float32)
            rwb = rw.astype(bf)
            A = in_b[slot].astype(bf)
            h1 = []
            for k in range(H):
                wa_k = w1_ref[m, k, 0].astype(bf)
                h1.append(jnp.maximum(A * wa_k + rwb[:, k:k + 1]
                                      + clT2[m][k:k + 1, :], bf(0.0)))
            h2 = []
            for k2 in range(H):
                acc = h1[0] * w2_ref[m, k2 * H].astype(bf)
                for j in range(1, H):
                    acc = acc + h1[j] * w2_ref[m, k2 * H + j].astype(bf)
                h2.append(jnp.maximum(acc + b2_ref[m, k2].astype(bf), bf(0.0)))
            out = h2[0] * w3_ref[m, 0].astype(bf)
            for j in range(1, H):
                out = out + h2[j] * w3_ref[m, j].astype(bf)
            out = out + b3_ref[m, 0].astype(bf)
            out_s[rows, :] = out
            outT_s[:, rows] = out.T
        tpart = jnp.sum(tb[slot], axis=0, keepdims=True)
        tgacc = tpart if tgacc is None else tgacc + tpart

    x = y_ref[...]
    for l in range(L):
        xb = x.astype(bf)
        p0 = jnp.dot(a_s[...], xb, preferred_element_type=jnp.float32)
        p1 = jnp.dot(aT_s[...], xb, preferred_element_type=jnp.float32)
        p2 = jnp.dot(ad_s[...], xb, preferred_element_type=jnp.float32)
        p3 = jnp.dot(adT_s[...], xb, preferred_element_type=jnp.float32)
        x = (jnp.dot(p0, W_ref[l, 0], preferred_element_type=jnp.float32)
             + jnp.dot(p1, W_ref[l, 1], preferred_element_type=jnp.float32)
             + jnp.dot(p2, W_ref[l, 2], preferred_element_type=jnp.float32)
             + jnp.dot(p3, W_ref[l, 3], preferred_element_type=jnp.float32)
             + jnp.dot(x, W_ref[l, 4], preferred_element_type=jnp.float32)
             + b_ref[l].reshape(1, D))
        if l < L - 1:
            x = jnp.maximum(x, 0.0)
    ones8 = jnp.ones((8, D), jnp.float32)
    tg8 = jnp.broadcast_to(tgacc * (1.0 / N), (8, N))
    tgm = lax.dot_general(tg8, ones8, (((0,), (0,)), ((), ())),
                          preferred_element_type=jnp.float32) * 0.125
    o_ref[...] = x * tgm


def kernel(y, adj, adj_deriv, t_grad, idx_emb, msg_W1, msg_b1, msg_W2, msg_b2,
           msg_W3, msg_b3, gnn_W, gnn_b):
    smem = pl.BlockSpec(memory_space=pltpu.SMEM)
    hbm = pl.BlockSpec(memory_space=pl.ANY)
    out = pl.pallas_call(
        _fused_kernel,
        in_specs=[
            smem, smem, smem, smem, smem,
            hbm, hbm, hbm,
            pl.BlockSpec((N, IDX), lambda: (0, 0)),
            pl.BlockSpec((2, H, 2 * IDX + 1), lambda: (0, 0, 0)),
            pl.BlockSpec((N, D), lambda: (0, 0)),
            pl.BlockSpec((L, 5, D, D), lambda: (0, 0, 0, 0)),
            pl.BlockSpec((L, D), lambda: (0, 0)),
        ],
        out_specs=pl.BlockSpec((N, D), lambda: (0, 0)),
        out_shape=jax.ShapeDtypeStruct((N, D), jnp.float32),
        scratch_shapes=[
            pltpu.VMEM((N, N), jnp.bfloat16),
            pltpu.VMEM((N, N), jnp.bfloat16),
            pltpu.VMEM((N, N), jnp.bfloat16),
            pltpu.VMEM((N, N), jnp.bfloat16),
            pltpu.VMEM((2, BR, N), jnp.float32),
            pltpu.VMEM((2, BR, N), jnp.float32),
            pltpu.VMEM((2, BR, N), jnp.float32),
            pltpu.SemaphoreType.DMA((3, 2)),
        ],
        name="fused_msg_gnn",
    )(msg_b1, msg_W2.reshape(2, H * H), msg_b2, msg_W3.reshape(2, H), msg_b3,
      adj, adj_deriv, t_grad, idx_emb, msg_W1, y, gnn_W, gnn_b)
    return out
